# Initial kernel scaffold; baseline (speedup 1.0000x reference)
import jax
import jax.numpy as jnp
from jax import lax
from jax.experimental import pallas as pl
from jax.experimental.pallas import tpu as pltpu, tpu_sc as plsc

VIEWS = 180
NDETU = 512
NVX = 256
NVY = 256
EXTENT = 2
B = 2
SINO = VIEWS * NDETU
SEG = VIEWS * EXTENT
NVOX = NVX * NVY
SCALE = (3.141592653589793 - 0.0) / (2 * VIEWS * EXTENT)

NW = 32
VOX_PER_W = NVOX // NW
CHUNK_VOX = 16
CHUNK_EL = CHUNK_VOX * SEG
NCHUNK = VOX_PER_W // CHUNK_VOX


def _body(table_hbm, idx_hbm, w_hbm, bias_hbm, out_hbm,
          table_v, idx_v, w_v, bias_v, out0_v, out1_v):
    wid = lax.axis_index("s") * 2 + lax.axis_index("c")
    v0 = wid * VOX_PER_W
    e0 = v0 * SEG

    pltpu.sync_copy(table_hbm, table_v)
    pltpu.sync_copy(bias_hbm.at[pl.ds(v0, VOX_PER_W)], bias_v)

    lanes = lax.iota(jnp.int32, 16)
    pos_base = lanes * SEG
    zero = jnp.zeros((16,), jnp.float32)
    himask = jnp.full((16,), -65536, jnp.int32)

    @pl.loop(0, NCHUNK)
    def _chunks(c):
        pltpu.sync_copy(idx_hbm.at[pl.ds(e0 + c * CHUNK_EL, CHUNK_EL)], idx_v)
        pltpu.sync_copy(w_hbm.at[pl.ds(e0 + c * CHUNK_EL, CHUNK_EL)], w_v)

        @pl.loop(0, SEG, init_carry=(zero, zero), unroll=4)
        def accs(j, carry):
            a0, a1 = carry
            pos = pos_base + j
            si = plsc.load_gather(idx_v, [pos])
            g = plsc.load_gather(table_v, [si])
            w = plsc.load_gather(w_v, [pos])
            x0 = plsc.bitcast(g << 16, jnp.float32)
            x1 = plsc.bitcast(g & himask, jnp.float32)
            return a0 + x0 * w, a1 + x1 * w

        acc0, acc1 = accs
        l0 = c * CHUNK_VOX
        bv = bias_v[pl.ds(l0, CHUNK_VOX)]
        o0 = lax.rev(acc0 * SCALE + bv, (0,))
        o1 = lax.rev(acc1 * SCALE + bv, (0,))
        ro = VOX_PER_W - CHUNK_VOX - l0
        out0_v[pl.ds(ro, CHUNK_VOX)] = o0
        out1_v[pl.ds(ro, CHUNK_VOX)] = o1

    fo = NVOX - v0 - VOX_PER_W
    pltpu.sync_copy(out0_v, out_hbm.at[pl.ds(fo, VOX_PER_W)])
    pltpu.sync_copy(out1_v, out_hbm.at[pl.ds(NVOX + fo, VOX_PER_W)])


_mesh = plsc.VectorSubcoreMesh(core_axis_name="c", subcore_axis_name="s")

_sc_call = pl.kernel(
    _body,
    out_type=jax.ShapeDtypeStruct((B * NVOX,), jnp.float32),
    mesh=_mesh,
    scratch_types=[
        pltpu.VMEM((SINO,), jnp.int32),
        pltpu.VMEM((CHUNK_EL,), jnp.int32),
        pltpu.VMEM((CHUNK_EL,), jnp.float32),
        pltpu.VMEM((VOX_PER_W,), jnp.float32),
        pltpu.VMEM((VOX_PER_W,), jnp.float32),
        pltpu.VMEM((VOX_PER_W,), jnp.float32),
    ],
)


@jax.jit
def kernel(input, weight, bias, indices):
    x = input.reshape(B, SINO)
    lo = lax.bitcast_convert_type(x[0].astype(jnp.bfloat16), jnp.uint16)
    hi = lax.bitcast_convert_type(x[1].astype(jnp.bfloat16), jnp.uint16)
    table = (lo.astype(jnp.uint32) | (hi.astype(jnp.uint32) << 16)
             ).view(jnp.int32)
    out = _sc_call(table, indices, weight.reshape(-1), bias)
    return out.reshape(B, 1, NVX, NVY)


# speedup vs baseline: 280.6565x; 280.6565x over previous
import jax
import jax.numpy as jnp
from jax import lax
from jax.experimental import pallas as pl
from jax.experimental.pallas import tpu as pltpu, tpu_sc as plsc

VIEWS = 180
NDETU = 512
NVX = 256
NVY = 256
EXTENT = 2
B = 2
SINO = VIEWS * NDETU
SEG = VIEWS * EXTENT
NVOX = NVX * NVY
SCALE = (3.141592653589793 - 0.0) / (2 * VIEWS * EXTENT)

NW = 32
VOX_PER_W = NVOX // NW
CHUNK_VOX = 16
CHUNK_EL = CHUNK_VOX * SEG
NCHUNK = VOX_PER_W // CHUNK_VOX


def _body(table_hbm, idx_hbm, w_hbm, bias_hbm, out_hbm,
          table_v, idx_v, w_v, bias_v, out0_v, out1_v):
    wid = lax.axis_index("s") * 2 + lax.axis_index("c")
    v0 = wid * VOX_PER_W
    e0 = v0 * SEG

    pltpu.sync_copy(table_hbm, table_v)
    pltpu.sync_copy(bias_hbm.at[pl.ds(v0, VOX_PER_W)], bias_v)

    lanes = lax.iota(jnp.int32, 16)
    pos_base = lanes * SEG
    zero = jnp.zeros((16,), jnp.float32)
    himask = jnp.full((16,), -65536, jnp.int32)

    @pl.loop(0, NCHUNK)
    def _chunks(c):
        pltpu.sync_copy(idx_hbm.at[pl.ds(e0 + c * CHUNK_EL, CHUNK_EL)], idx_v)
        pltpu.sync_copy(w_hbm.at[pl.ds(e0 + c * CHUNK_EL, CHUNK_EL)], w_v)

        @pl.loop(0, SEG, init_carry=(zero, zero), unroll=4)
        def accs(j, carry):
            a0, a1 = carry
            pos = pos_base + j
            si = plsc.load_gather(idx_v, [pos])
            g = plsc.load_gather(table_v, [si])
            w = plsc.load_gather(w_v, [pos])
            x0 = plsc.bitcast(g << 16, jnp.float32)
            x1 = plsc.bitcast(g & himask, jnp.float32)
            return a0 + x0 * w, a1 + x1 * w

        acc0, acc1 = accs
        l0 = c * CHUNK_VOX
        bv = bias_v[pl.ds(l0, CHUNK_VOX)]
        o0 = lax.rev(acc0 * SCALE + bv, (0,))
        o1 = lax.rev(acc1 * SCALE + bv, (0,))
        ro = VOX_PER_W - CHUNK_VOX - l0
        out0_v[pl.ds(ro, CHUNK_VOX)] = o0
        out1_v[pl.ds(ro, CHUNK_VOX)] = o1

    fo = NVOX - v0 - VOX_PER_W
    pltpu.sync_copy(out0_v, out_hbm.at[pl.ds(fo, VOX_PER_W)])
    pltpu.sync_copy(out1_v, out_hbm.at[pl.ds(NVOX + fo, VOX_PER_W)])


_mesh = plsc.VectorSubcoreMesh(core_axis_name="c", subcore_axis_name="s")

_sc_call = pl.kernel(
    _body,
    out_type=jax.ShapeDtypeStruct((B * NVOX,), jnp.float32),
    mesh=_mesh,
    compiler_params=pltpu.CompilerParams(needs_layout_passes=False),
    scratch_types=[
        pltpu.VMEM((SINO,), jnp.int32),
        pltpu.VMEM((CHUNK_EL,), jnp.int32),
        pltpu.VMEM((CHUNK_EL,), jnp.float32),
        pltpu.VMEM((VOX_PER_W,), jnp.float32),
        pltpu.VMEM((VOX_PER_W,), jnp.float32),
        pltpu.VMEM((VOX_PER_W,), jnp.float32),
    ],
)


@jax.jit
def kernel(input, weight, bias, indices):
    x = input.reshape(B, SINO)
    lo = lax.bitcast_convert_type(x[0].astype(jnp.bfloat16), jnp.uint16)
    hi = lax.bitcast_convert_type(x[1].astype(jnp.bfloat16), jnp.uint16)
    table = (lo.astype(jnp.uint32) | (hi.astype(jnp.uint32) << 16)
             ).view(jnp.int32)
    out = _sc_call(table, indices, weight.reshape(-1), bias)
    return out.reshape(B, 1, NVX, NVY)


# speedup vs baseline: 617.1234x; 2.1989x over previous
import jax
import jax.numpy as jnp
from jax import lax
from jax.experimental import pallas as pl
from jax.experimental.pallas import tpu as pltpu, tpu_sc as plsc

VIEWS = 180
NDETU = 512
NVX = 256
NVY = 256
EXTENT = 2
B = 2
SINO = VIEWS * NDETU
SEG = VIEWS * EXTENT
NVOX = NVX * NVY
SCALE = (3.141592653589793 - 0.0) / (2 * VIEWS * EXTENT)

NW = 32
VOX_PER_W = NVOX // NW
CHUNK_VOX = 16
CHUNK_EL = CHUNK_VOX * SEG
NCHUNK = VOX_PER_W // CHUNK_VOX


def _body(table_hbm, idx_hbm, w_hbm, bias_hbm, out_hbm,
          table_v, idx_v0, idx_v1, w_v0, w_v1, bias_v, out0_v, out1_v,
          sem0, sem1):
    wid = lax.axis_index("s") * 2 + lax.axis_index("c")
    v0 = wid * VOX_PER_W
    e0 = v0 * SEG

    pltpu.sync_copy(table_hbm, table_v)
    pltpu.sync_copy(bias_hbm.at[pl.ds(v0, VOX_PER_W)], bias_v)

    lanes = lax.iota(jnp.int32, 16)
    pos_base = lanes * SEG
    zero = jnp.zeros((16,), jnp.float32)
    himask = jnp.full((16,), -65536, jnp.int32)

    def fetch(c, ibuf, wbuf, sem):
        pltpu.async_copy(idx_hbm.at[pl.ds(e0 + c * CHUNK_EL, CHUNK_EL)],
                         ibuf, sem)
        pltpu.async_copy(w_hbm.at[pl.ds(e0 + c * CHUNK_EL, CHUNK_EL)],
                         wbuf, sem)

    def drain(ibuf, wbuf, sem):
        pltpu.make_async_copy(idx_hbm.at[pl.ds(0, CHUNK_EL)], ibuf, sem).wait()
        pltpu.make_async_copy(w_hbm.at[pl.ds(0, CHUNK_EL)], wbuf, sem).wait()

    def compute(c, ibuf, wbuf):
        @pl.loop(0, SEG, init_carry=(zero, zero), unroll=4)
        def accs(j, carry):
            a0, a1 = carry
            pos = pos_base + j
            si = plsc.load_gather(ibuf, [pos])
            g = plsc.load_gather(table_v, [si])
            w = plsc.load_gather(wbuf, [pos])
            x0 = plsc.bitcast(g << 16, jnp.float32)
            x1 = plsc.bitcast(g & himask, jnp.float32)
            return a0 + x0 * w, a1 + x1 * w

        acc0, acc1 = accs
        l0 = c * CHUNK_VOX
        bv = bias_v[pl.ds(l0, CHUNK_VOX)]
        o0 = lax.rev(acc0 * SCALE + bv, (0,))
        o1 = lax.rev(acc1 * SCALE + bv, (0,))
        ro = VOX_PER_W - CHUNK_VOX - l0
        out0_v[pl.ds(ro, CHUNK_VOX)] = o0
        out1_v[pl.ds(ro, CHUNK_VOX)] = o1

    fetch(0, idx_v0, w_v0, sem0)

    @pl.loop(0, NCHUNK // 2)
    def _pairs(t):
        c = 2 * t
        fetch(c + 1, idx_v1, w_v1, sem1)
        drain(idx_v0, w_v0, sem0)
        compute(c, idx_v0, w_v0)

        @pl.when(c + 2 < NCHUNK)
        def _():
            fetch(c + 2, idx_v0, w_v0, sem0)

        drain(idx_v1, w_v1, sem1)
        compute(c + 1, idx_v1, w_v1)

    fo = NVOX - v0 - VOX_PER_W
    pltpu.sync_copy(out0_v, out_hbm.at[pl.ds(fo, VOX_PER_W)])
    pltpu.sync_copy(out1_v, out_hbm.at[pl.ds(NVOX + fo, VOX_PER_W)])


_mesh = plsc.VectorSubcoreMesh(core_axis_name="c", subcore_axis_name="s")

_sc_call = pl.kernel(
    _body,
    out_type=jax.ShapeDtypeStruct((B * NVOX,), jnp.float32),
    mesh=_mesh,
    compiler_params=pltpu.CompilerParams(needs_layout_passes=False),
    scratch_types=[
        pltpu.VMEM((SINO,), jnp.int32),
        pltpu.VMEM((CHUNK_EL,), jnp.int32),
        pltpu.VMEM((CHUNK_EL,), jnp.int32),
        pltpu.VMEM((CHUNK_EL,), jnp.float32),
        pltpu.VMEM((CHUNK_EL,), jnp.float32),
        pltpu.VMEM((VOX_PER_W,), jnp.float32),
        pltpu.VMEM((VOX_PER_W,), jnp.float32),
        pltpu.VMEM((VOX_PER_W,), jnp.float32),
        pltpu.SemaphoreType.DMA,
        pltpu.SemaphoreType.DMA,
    ],
)


@jax.jit
def kernel(input, weight, bias, indices):
    x = input.reshape(B, SINO)
    lo = lax.bitcast_convert_type(x[0].astype(jnp.bfloat16), jnp.uint16)
    hi = lax.bitcast_convert_type(x[1].astype(jnp.bfloat16), jnp.uint16)
    table = (lo.astype(jnp.uint32) | (hi.astype(jnp.uint32) << 16)
             ).view(jnp.int32)
    out = _sc_call(table, indices, weight.reshape(-1), bias)
    return out.reshape(B, 1, NVX, NVY)


# speedup vs baseline: 665.1065x; 1.0778x over previous
import jax
import jax.numpy as jnp
from jax import lax
from jax.experimental import pallas as pl
from jax.experimental.pallas import tpu as pltpu, tpu_sc as plsc

VIEWS = 180
NDETU = 512
NVX = 256
NVY = 256
EXTENT = 2
B = 2
SINO = VIEWS * NDETU
SEG = VIEWS * EXTENT
NVOX = NVX * NVY
SCALE = (3.141592653589793 - 0.0) / (2 * VIEWS * EXTENT)

NW = 32
VOX_PER_W = NVOX // NW
CHUNK_VOX = 16
CHUNK_EL = CHUNK_VOX * SEG
NCHUNK = VOX_PER_W // CHUNK_VOX


NBUF = 3


def _body(table_hbm, idx_hbm, w_hbm, bias_hbm, out_hbm,
          table_v, idx_v0, idx_v1, idx_v2, w_v0, w_v1, w_v2,
          bias_v, out0_v, out1_v, tsem, sem0, sem1, sem2):
    wid = lax.axis_index("s") * 2 + lax.axis_index("c")
    v0 = wid * VOX_PER_W
    e0 = v0 * SEG

    ibufs = (idx_v0, idx_v1, idx_v2)
    wbufs = (w_v0, w_v1, w_v2)
    sems = (sem0, sem1, sem2)

    lanes = lax.iota(jnp.int32, 16)
    zero = jnp.zeros((16,), jnp.float32)
    himask = jnp.full((16,), -65536, jnp.int32)

    def fetch(c, s):
        pltpu.async_copy(idx_hbm.at[pl.ds(e0 + c * CHUNK_EL, CHUNK_EL)],
                         ibufs[s], sems[s])
        pltpu.async_copy(w_hbm.at[pl.ds(e0 + c * CHUNK_EL, CHUNK_EL)],
                         wbufs[s], sems[s])
        pltpu.async_copy(bias_hbm.at[pl.ds(v0 + c * CHUNK_VOX, CHUNK_VOX)],
                         bias_v.at[pl.ds(s * CHUNK_VOX, CHUNK_VOX)], sems[s])

    def consume(c, s):
        pltpu.make_async_copy(idx_hbm.at[pl.ds(0, CHUNK_EL)],
                              ibufs[s], sems[s]).wait()
        pltpu.make_async_copy(w_hbm.at[pl.ds(0, CHUNK_EL)],
                              wbufs[s], sems[s]).wait()
        pltpu.make_async_copy(bias_hbm.at[pl.ds(0, CHUNK_VOX)],
                              bias_v.at[pl.ds(s * CHUNK_VOX, CHUNK_VOX)],
                              sems[s]).wait()

        @pl.loop(0, SEG, init_carry=(zero, zero, lanes * SEG), unroll=4)
        def accs(j, carry):
            a0, a1, pos = carry
            si = plsc.load_gather(ibufs[s], [pos])
            g = plsc.load_gather(table_v, [si])
            w = plsc.load_gather(wbufs[s], [pos])
            x0 = plsc.bitcast(g << 16, jnp.float32)
            x1 = plsc.bitcast(g & himask, jnp.float32)
            return a0 + x0 * w, a1 + x1 * w, pos + 1

        acc0, acc1, _ = accs
        bv = bias_v[pl.ds(s * CHUNK_VOX, CHUNK_VOX)]
        o0 = lax.rev(acc0 * SCALE + bv, (0,))
        o1 = lax.rev(acc1 * SCALE + bv, (0,))
        ro = VOX_PER_W - CHUNK_VOX - c * CHUNK_VOX
        out0_v[pl.ds(ro, CHUNK_VOX)] = o0
        out1_v[pl.ds(ro, CHUNK_VOX)] = o1

    tcopy = pltpu.async_copy(table_hbm, table_v, tsem)
    fetch(0, 0)
    fetch(1, 1)
    tcopy.wait()

    assert (NCHUNK - 2) % NBUF == 0

    @pl.loop(0, (NCHUNK - 2) // NBUF)
    def _ring(t):
        c = NBUF * t
        for b in range(NBUF):
            fetch(c + b + 2, (b + 2) % NBUF)
            consume(c + b, b)

    c = NCHUNK - 2
    consume(c, c % NBUF)
    consume(c + 1, (c + 1) % NBUF)

    fo = NVOX - v0 - VOX_PER_W
    pltpu.sync_copy(out0_v, out_hbm.at[pl.ds(fo, VOX_PER_W)])
    pltpu.sync_copy(out1_v, out_hbm.at[pl.ds(NVOX + fo, VOX_PER_W)])


_mesh = plsc.VectorSubcoreMesh(core_axis_name="c", subcore_axis_name="s")

_sc_call = pl.kernel(
    _body,
    out_type=jax.ShapeDtypeStruct((B * NVOX,), jnp.float32),
    mesh=_mesh,
    compiler_params=pltpu.CompilerParams(needs_layout_passes=False),
    scratch_types=[
        pltpu.VMEM((SINO,), jnp.int32),
        pltpu.VMEM((CHUNK_EL,), jnp.int32),
        pltpu.VMEM((CHUNK_EL,), jnp.int32),
        pltpu.VMEM((CHUNK_EL,), jnp.int32),
        pltpu.VMEM((CHUNK_EL,), jnp.float32),
        pltpu.VMEM((CHUNK_EL,), jnp.float32),
        pltpu.VMEM((CHUNK_EL,), jnp.float32),
        pltpu.VMEM((NBUF * CHUNK_VOX,), jnp.float32),
        pltpu.VMEM((VOX_PER_W,), jnp.float32),
        pltpu.VMEM((VOX_PER_W,), jnp.float32),
        pltpu.SemaphoreType.DMA,
        pltpu.SemaphoreType.DMA,
        pltpu.SemaphoreType.DMA,
        pltpu.SemaphoreType.DMA,
    ],
)


@jax.jit
def kernel(input, weight, bias, indices):
    x = input.reshape(B, SINO)
    lo = lax.bitcast_convert_type(x[0].astype(jnp.bfloat16), jnp.uint16)
    hi = lax.bitcast_convert_type(x[1].astype(jnp.bfloat16), jnp.uint16)
    table = (lo.astype(jnp.uint32) | (hi.astype(jnp.uint32) << 16)
             ).view(jnp.int32)
    out = _sc_call(table, indices, weight.reshape(-1), bias)
    return out.reshape(B, 1, NVX, NVY)


# speedup vs baseline: 665.9023x; 1.0012x over previous
import jax
import jax.numpy as jnp
from jax import lax
from jax.experimental import pallas as pl
from jax.experimental.pallas import tpu as pltpu, tpu_sc as plsc

VIEWS = 180
NDETU = 512
NVX = 256
NVY = 256
EXTENT = 2
B = 2
SINO = VIEWS * NDETU
SEG = VIEWS * EXTENT
NVOX = NVX * NVY
SCALE = (3.141592653589793 - 0.0) / (2 * VIEWS * EXTENT)

NW = 32
VOX_PER_W = NVOX // NW
CHUNK_VOX = 16
CHUNK_EL = CHUNK_VOX * SEG
NCHUNK = VOX_PER_W // CHUNK_VOX


NBUF = 3


def _body(table_hbm, idx_hbm, w_hbm, bias_hbm, out_hbm,
          table_v, idx_v0, idx_v1, idx_v2, w_v0, w_v1, w_v2,
          bias_v, out0_v, out1_v, tsem, sem0, sem1, sem2):
    wid = lax.axis_index("s") * 2 + lax.axis_index("c")
    v0 = wid * VOX_PER_W
    e0 = v0 * SEG

    ibufs = (idx_v0, idx_v1, idx_v2)
    wbufs = (w_v0, w_v1, w_v2)
    sems = (sem0, sem1, sem2)

    lanes = lax.iota(jnp.int32, 16)
    zero = jnp.zeros((16,), jnp.float32)
    himask = jnp.full((16,), -65536, jnp.int32)

    def fetch(c, s):
        pltpu.async_copy(idx_hbm.at[pl.ds(e0 + c * CHUNK_EL, CHUNK_EL)],
                         ibufs[s], sems[s])
        pltpu.async_copy(w_hbm.at[pl.ds(e0 + c * CHUNK_EL, CHUNK_EL)],
                         wbufs[s], sems[s])
        pltpu.async_copy(bias_hbm.at[pl.ds(v0 + c * CHUNK_VOX, CHUNK_VOX)],
                         bias_v.at[pl.ds(s * CHUNK_VOX, CHUNK_VOX)], sems[s])

    def consume(c, s):
        pltpu.make_async_copy(idx_hbm.at[pl.ds(0, CHUNK_EL)],
                              ibufs[s], sems[s]).wait()
        pltpu.make_async_copy(w_hbm.at[pl.ds(0, CHUNK_EL)],
                              wbufs[s], sems[s]).wait()
        pltpu.make_async_copy(bias_hbm.at[pl.ds(0, CHUNK_VOX)],
                              bias_v.at[pl.ds(s * CHUNK_VOX, CHUNK_VOX)],
                              sems[s]).wait()

        @pl.loop(0, SEG, init_carry=(zero, zero, lanes * SEG), unroll=8)
        def accs(j, carry):
            a0, a1, pos = carry
            si = plsc.load_gather(ibufs[s], [pos])
            g = plsc.load_gather(table_v, [si])
            w = plsc.load_gather(wbufs[s], [pos])
            x0 = plsc.bitcast(g << 16, jnp.float32)
            x1 = plsc.bitcast(g & himask, jnp.float32)
            return a0 + x0 * w, a1 + x1 * w, pos + 1

        acc0, acc1, _ = accs
        bv = bias_v[pl.ds(s * CHUNK_VOX, CHUNK_VOX)]
        o0 = lax.rev(acc0 * SCALE + bv, (0,))
        o1 = lax.rev(acc1 * SCALE + bv, (0,))
        ro = VOX_PER_W - CHUNK_VOX - c * CHUNK_VOX
        out0_v[pl.ds(ro, CHUNK_VOX)] = o0
        out1_v[pl.ds(ro, CHUNK_VOX)] = o1

    tcopy = pltpu.async_copy(table_hbm, table_v, tsem)
    fetch(0, 0)
    fetch(1, 1)
    tcopy.wait()

    assert (NCHUNK - 2) % NBUF == 0

    @pl.loop(0, (NCHUNK - 2) // NBUF)
    def _ring(t):
        c = NBUF * t
        for b in range(NBUF):
            fetch(c + b + 2, (b + 2) % NBUF)
            consume(c + b, b)

    c = NCHUNK - 2
    consume(c, c % NBUF)
    consume(c + 1, (c + 1) % NBUF)

    fo = NVOX - v0 - VOX_PER_W
    pltpu.sync_copy(out0_v, out_hbm.at[pl.ds(fo, VOX_PER_W)])
    pltpu.sync_copy(out1_v, out_hbm.at[pl.ds(NVOX + fo, VOX_PER_W)])


_mesh = plsc.VectorSubcoreMesh(core_axis_name="c", subcore_axis_name="s")

_sc_call = pl.kernel(
    _body,
    out_type=jax.ShapeDtypeStruct((B * NVOX,), jnp.float32),
    mesh=_mesh,
    compiler_params=pltpu.CompilerParams(needs_layout_passes=False),
    scratch_types=[
        pltpu.VMEM((SINO,), jnp.int32),
        pltpu.VMEM((CHUNK_EL,), jnp.int32),
        pltpu.VMEM((CHUNK_EL,), jnp.int32),
        pltpu.VMEM((CHUNK_EL,), jnp.int32),
        pltpu.VMEM((CHUNK_EL,), jnp.float32),
        pltpu.VMEM((CHUNK_EL,), jnp.float32),
        pltpu.VMEM((CHUNK_EL,), jnp.float32),
        pltpu.VMEM((NBUF * CHUNK_VOX,), jnp.float32),
        pltpu.VMEM((VOX_PER_W,), jnp.float32),
        pltpu.VMEM((VOX_PER_W,), jnp.float32),
        pltpu.SemaphoreType.DMA,
        pltpu.SemaphoreType.DMA,
        pltpu.SemaphoreType.DMA,
        pltpu.SemaphoreType.DMA,
    ],
)


@jax.jit
def kernel(input, weight, bias, indices):
    x = input.reshape(B, SINO)
    lo = lax.bitcast_convert_type(x[0].astype(jnp.bfloat16), jnp.uint16)
    hi = lax.bitcast_convert_type(x[1].astype(jnp.bfloat16), jnp.uint16)
    table = (lo.astype(jnp.uint32) | (hi.astype(jnp.uint32) << 16)
             ).view(jnp.int32)
    out = _sc_call(table, indices, weight.reshape(-1), bias)
    return out.reshape(B, 1, NVX, NVY)


# speedup vs baseline: 709.9961x; 1.0662x over previous
import jax
import jax.numpy as jnp
from jax import lax
from jax.experimental import pallas as pl
from jax.experimental.pallas import tpu as pltpu, tpu_sc as plsc

VIEWS = 180
NDETU = 512
NVX = 256
NVY = 256
EXTENT = 2
B = 2
SINO = VIEWS * NDETU
SEG = VIEWS * EXTENT
NVOX = NVX * NVY
SCALE = (3.141592653589793 - 0.0) / (2 * VIEWS * EXTENT)

NW = 32
VOX_PER_W = NVOX // NW
CHUNK_VOX = 16
CHUNK_EL = CHUNK_VOX * SEG
NCHUNK = VOX_PER_W // CHUNK_VOX


NBUF = 3


def _body(table_hbm, idx_hbm, w_hbm, bias_hbm, out_hbm,
          table_v, idx_v0, idx_v1, idx_v2, w_v0, w_v1, w_v2,
          out0_v, out1_v, tsem, sem0, sem1, sem2):
    wid = lax.axis_index("s") * 2 + lax.axis_index("c")
    v0 = wid * VOX_PER_W
    e0 = v0 * SEG

    ibufs = (idx_v0, idx_v1, idx_v2)
    wbufs = (w_v0, w_v1, w_v2)
    sems = (sem0, sem1, sem2)

    lanes = lax.iota(jnp.int32, 16)
    zero = jnp.zeros((16,), jnp.float32)
    himask = jnp.full((16,), -65536, jnp.int32)

    maskLo = jnp.where(lanes < 8, 1.0, 0.0).astype(jnp.float32)
    maskHi = jnp.where(lanes < 8, 0.0, 1.0).astype(jnp.float32)

    def fetch(c, s):
        pltpu.async_copy(idx_hbm.at[pl.ds(e0 + c * CHUNK_EL, CHUNK_EL)],
                         ibufs[s], sems[s])
        pltpu.async_copy(w_hbm.at[pl.ds(e0 + c * CHUNK_EL, CHUNK_EL)],
                         wbufs[s], sems[s])

    def triple(s, off):
        si = ibufs[s][pl.ds(off, 16)]
        g = plsc.load_gather(table_v, [si])
        w = wbufs[s][pl.ds(off, 16)]
        x0 = plsc.bitcast(g << 16, jnp.float32)
        x1 = plsc.bitcast(g & himask, jnp.float32)
        return x0 * w, x1 * w

    def mac(s, base, n, a0_init, a1_init):
        @pl.loop(0, n, init_carry=(a0_init, a1_init), unroll=11)
        def r(k, carry):
            a0, a1 = carry
            p0, p1 = triple(s, base + k * 16)
            return a0 + p0, a1 + p1

        return r

    def consume(c, s):
        pltpu.make_async_copy(idx_hbm.at[pl.ds(0, CHUNK_EL)],
                              ibufs[s], sems[s]).wait()
        pltpu.make_async_copy(w_hbm.at[pl.ds(0, CHUNK_EL)],
                              wbufs[s], sems[s]).wait()

        @pl.loop(0, CHUNK_VOX // 2, init_carry=(zero, zero))
        def pairs(p, carry):
            outv0, outv1 = carry
            eb = p * (2 * SEG)
            a0, a1 = mac(s, eb, 22, zero, zero)
            xw0, xw1 = triple(s, eb + 352)
            a0 = a0 + xw0 * maskLo
            a1 = a1 + xw1 * maskLo
            b0, b1 = mac(s, eb + 368, 22, xw0 * maskHi, xw1 * maskHi)
            outv0 = jnp.where(lanes == 2 * p, jnp.sum(a0), outv0)
            outv0 = jnp.where(lanes == 2 * p + 1, jnp.sum(b0), outv0)
            outv1 = jnp.where(lanes == 2 * p, jnp.sum(a1), outv1)
            outv1 = jnp.where(lanes == 2 * p + 1, jnp.sum(b1), outv1)
            return outv0, outv1

        outv0, outv1 = pairs
        ro = VOX_PER_W - CHUNK_VOX - c * CHUNK_VOX
        out0_v[pl.ds(ro, CHUNK_VOX)] = (out0_v[pl.ds(ro, CHUNK_VOX)]
                                        + lax.rev(outv0, (0,)) * SCALE)
        out1_v[pl.ds(ro, CHUNK_VOX)] = (out1_v[pl.ds(ro, CHUNK_VOX)]
                                        + lax.rev(outv1, (0,)) * SCALE)

    tcopy = pltpu.async_copy(table_hbm, table_v, tsem)

    pltpu.sync_copy(bias_hbm.at[pl.ds(v0, VOX_PER_W)],
                    w_v0.at[pl.ds(0, VOX_PER_W)])

    @pl.loop(0, NCHUNK)
    def _stage(k):
        rv = lax.rev(w_v0[pl.ds(k * 16, 16)], (0,))
        out0_v[pl.ds(VOX_PER_W - 16 - k * 16, 16)] = rv
        out1_v[pl.ds(VOX_PER_W - 16 - k * 16, 16)] = rv

    fetch(0, 0)
    fetch(1, 1)
    tcopy.wait()

    assert (NCHUNK - 2) % NBUF == 0

    @pl.loop(0, (NCHUNK - 2) // NBUF)
    def _ring(t):
        c = NBUF * t
        for b in range(NBUF):
            fetch(c + b + 2, (b + 2) % NBUF)
            consume(c + b, b)

    c = NCHUNK - 2
    consume(c, c % NBUF)
    consume(c + 1, (c + 1) % NBUF)

    fo = NVOX - v0 - VOX_PER_W
    pltpu.sync_copy(out0_v, out_hbm.at[pl.ds(fo, VOX_PER_W)])
    pltpu.sync_copy(out1_v, out_hbm.at[pl.ds(NVOX + fo, VOX_PER_W)])


_mesh = plsc.VectorSubcoreMesh(core_axis_name="c", subcore_axis_name="s")

_sc_call = pl.kernel(
    _body,
    out_type=jax.ShapeDtypeStruct((B * NVOX,), jnp.float32),
    mesh=_mesh,
    compiler_params=pltpu.CompilerParams(needs_layout_passes=False),
    scratch_types=[
        pltpu.VMEM((SINO,), jnp.int32),
        pltpu.VMEM((CHUNK_EL,), jnp.int32),
        pltpu.VMEM((CHUNK_EL,), jnp.int32),
        pltpu.VMEM((CHUNK_EL,), jnp.int32),
        pltpu.VMEM((CHUNK_EL,), jnp.float32),
        pltpu.VMEM((CHUNK_EL,), jnp.float32),
        pltpu.VMEM((CHUNK_EL,), jnp.float32),
        pltpu.VMEM((VOX_PER_W,), jnp.float32),
        pltpu.VMEM((VOX_PER_W,), jnp.float32),
        pltpu.SemaphoreType.DMA,
        pltpu.SemaphoreType.DMA,
        pltpu.SemaphoreType.DMA,
        pltpu.SemaphoreType.DMA,
    ],
)


@jax.jit
def kernel(input, weight, bias, indices):
    x = input.reshape(B, SINO)
    lo = lax.bitcast_convert_type(x[0].astype(jnp.bfloat16), jnp.uint16)
    hi = lax.bitcast_convert_type(x[1].astype(jnp.bfloat16), jnp.uint16)
    table = (lo.astype(jnp.uint32) | (hi.astype(jnp.uint32) << 16)
             ).view(jnp.int32)
    out = _sc_call(table, indices, weight.reshape(-1), bias)
    return out.reshape(B, 1, NVX, NVY)


# speedup vs baseline: 734.6701x; 1.0348x over previous
import jax
import jax.numpy as jnp
from jax import lax
from jax.experimental import pallas as pl
from jax.experimental.pallas import tpu as pltpu, tpu_sc as plsc

VIEWS = 180
NDETU = 512
NVX = 256
NVY = 256
EXTENT = 2
B = 2
SINO = VIEWS * NDETU
SEG = VIEWS * EXTENT
NVOX = NVX * NVY
SCALE = (3.141592653589793 - 0.0) / (2 * VIEWS * EXTENT)

NW = 32
VOX_PER_W = NVOX // NW
CHUNK_VOX = 16
CHUNK_EL = CHUNK_VOX * SEG
NCHUNK = VOX_PER_W // CHUNK_VOX


NBUF = 3


def _body(table_hbm, idx_hbm, w_hbm, bias_hbm, out_hbm,
          table_v, idx_v0, idx_v1, idx_v2, w_v0, w_v1, w_v2,
          out0_v, out1_v, tsem, sem0, sem1, sem2):
    wid = lax.axis_index("s") * 2 + lax.axis_index("c")
    v0 = wid * VOX_PER_W
    e0 = v0 * SEG

    ibufs = (idx_v0, idx_v1, idx_v2)
    wbufs = (w_v0, w_v1, w_v2)
    sems = (sem0, sem1, sem2)

    lanes = lax.iota(jnp.int32, 16)
    zero = jnp.zeros((16,), jnp.float32)
    himask = jnp.full((16,), -65536, jnp.int32)

    maskLo = jnp.where(lanes < 8, 1.0, 0.0).astype(jnp.float32)
    maskHi = jnp.where(lanes < 8, 0.0, 1.0).astype(jnp.float32)
    base17 = lanes * 17

    def fetch(c, s):
        pltpu.async_copy(idx_hbm.at[pl.ds(e0 + c * CHUNK_EL, CHUNK_EL)],
                         ibufs[s], sems[s])
        pltpu.async_copy(w_hbm.at[pl.ds(e0 + c * CHUNK_EL, CHUNK_EL)],
                         wbufs[s], sems[s])

    def triple(s, off):
        si = ibufs[s][pl.ds(off, 16)]
        g = plsc.load_gather(table_v, [si])
        w = wbufs[s][pl.ds(off, 16)]
        x0 = plsc.bitcast(g << 16, jnp.float32)
        x1 = plsc.bitcast(g & himask, jnp.float32)
        return x0 * w, x1 * w

    def mac(s, base, n, a0_init, a1_init):
        @pl.loop(0, n, init_carry=(a0_init, a1_init), unroll=11)
        def r(k, carry):
            a0, a1 = carry
            p0, p1 = triple(s, base + k * 16)
            return a0 + p0, a1 + p1

        return r

    def consume(c, s):
        pltpu.make_async_copy(idx_hbm.at[pl.ds(0, CHUNK_EL)],
                              ibufs[s], sems[s]).wait()
        pltpu.make_async_copy(w_hbm.at[pl.ds(0, CHUNK_EL)],
                              wbufs[s], sems[s]).wait()

        @pl.loop(0, CHUNK_VOX // 2)
        def pairs(p):
            eb = p * (2 * SEG)
            a0, a1 = mac(s, eb, 22, zero, zero)
            xw0, xw1 = triple(s, eb + 352)
            a0 = a0 + xw0 * maskLo
            a1 = a1 + xw1 * maskLo
            b0, b1 = mac(s, eb + 368, 22, xw0 * maskHi, xw1 * maskHi)
            iv_even = base17 + 2 * p
            iv_odd = base17 + (2 * p + 1)
            plsc.store_scatter(ibufs[s], [iv_even],
                               plsc.bitcast(a0, jnp.int32))
            plsc.store_scatter(ibufs[s], [iv_odd],
                               plsc.bitcast(b0, jnp.int32))
            plsc.store_scatter(wbufs[s], [iv_even], a1)
            plsc.store_scatter(wbufs[s], [iv_odd], b1)

        outv0 = zero
        outv1 = zero
        for l in range(16):
            outv0 = outv0 + plsc.bitcast(ibufs[s][pl.ds(17 * l, 16)],
                                         jnp.float32)
            outv1 = outv1 + wbufs[s][pl.ds(17 * l, 16)]
        ro = VOX_PER_W - CHUNK_VOX - c * CHUNK_VOX
        out0_v[pl.ds(ro, CHUNK_VOX)] = (out0_v[pl.ds(ro, CHUNK_VOX)]
                                        + lax.rev(outv0, (0,)) * SCALE)
        out1_v[pl.ds(ro, CHUNK_VOX)] = (out1_v[pl.ds(ro, CHUNK_VOX)]
                                        + lax.rev(outv1, (0,)) * SCALE)

    tcopy = pltpu.async_copy(table_hbm, table_v, tsem)

    pltpu.sync_copy(bias_hbm.at[pl.ds(v0, VOX_PER_W)],
                    w_v0.at[pl.ds(0, VOX_PER_W)])

    @pl.loop(0, NCHUNK)
    def _stage(k):
        rv = lax.rev(w_v0[pl.ds(k * 16, 16)], (0,))
        out0_v[pl.ds(VOX_PER_W - 16 - k * 16, 16)] = rv
        out1_v[pl.ds(VOX_PER_W - 16 - k * 16, 16)] = rv

    fetch(0, 0)
    fetch(1, 1)
    tcopy.wait()

    assert (NCHUNK - 2) % NBUF == 0

    @pl.loop(0, (NCHUNK - 2) // NBUF)
    def _ring(t):
        c = NBUF * t
        for b in range(NBUF):
            fetch(c + b + 2, (b + 2) % NBUF)
            consume(c + b, b)

    c = NCHUNK - 2
    consume(c, c % NBUF)
    consume(c + 1, (c + 1) % NBUF)

    fo = NVOX - v0 - VOX_PER_W
    pltpu.sync_copy(out0_v, out_hbm.at[pl.ds(fo, VOX_PER_W)])
    pltpu.sync_copy(out1_v, out_hbm.at[pl.ds(NVOX + fo, VOX_PER_W)])


_mesh = plsc.VectorSubcoreMesh(core_axis_name="c", subcore_axis_name="s")

_sc_call = pl.kernel(
    _body,
    out_type=jax.ShapeDtypeStruct((B * NVOX,), jnp.float32),
    mesh=_mesh,
    compiler_params=pltpu.CompilerParams(needs_layout_passes=False),
    scratch_types=[
        pltpu.VMEM((SINO,), jnp.int32),
        pltpu.VMEM((CHUNK_EL,), jnp.int32),
        pltpu.VMEM((CHUNK_EL,), jnp.int32),
        pltpu.VMEM((CHUNK_EL,), jnp.int32),
        pltpu.VMEM((CHUNK_EL,), jnp.float32),
        pltpu.VMEM((CHUNK_EL,), jnp.float32),
        pltpu.VMEM((CHUNK_EL,), jnp.float32),
        pltpu.VMEM((VOX_PER_W,), jnp.float32),
        pltpu.VMEM((VOX_PER_W,), jnp.float32),
        pltpu.SemaphoreType.DMA,
        pltpu.SemaphoreType.DMA,
        pltpu.SemaphoreType.DMA,
        pltpu.SemaphoreType.DMA,
    ],
)


@jax.jit
def kernel(input, weight, bias, indices):
    x = input.reshape(B, SINO)
    lo = lax.bitcast_convert_type(x[0].astype(jnp.bfloat16), jnp.uint16)
    hi = lax.bitcast_convert_type(x[1].astype(jnp.bfloat16), jnp.uint16)
    table = (lo.astype(jnp.uint32) | (hi.astype(jnp.uint32) << 16)
             ).view(jnp.int32)
    out = _sc_call(table, indices, weight.reshape(-1), bias)
    return out.reshape(B, 1, NVX, NVY)


# speedup vs baseline: 741.0155x; 1.0086x over previous
import jax
import jax.numpy as jnp
from jax import lax
from jax.experimental import pallas as pl
from jax.experimental.pallas import tpu as pltpu, tpu_sc as plsc

VIEWS = 180
NDETU = 512
NVX = 256
NVY = 256
EXTENT = 2
B = 2
SINO = VIEWS * NDETU
SEG = VIEWS * EXTENT
NVOX = NVX * NVY
SCALE = (3.141592653589793 - 0.0) / (2 * VIEWS * EXTENT)

NW = 32
VOX_PER_W = NVOX // NW
CHUNK_VOX = 16
CHUNK_EL = CHUNK_VOX * SEG
NCHUNK = VOX_PER_W // CHUNK_VOX


NBUF = 3


def _body(table_hbm, idx_hbm, w_hbm, bias_hbm, out_hbm,
          table_v, idx_v0, idx_v1, idx_v2, w_v0, w_v1, w_v2,
          out0_v, out1_v, tsem, sem0, sem1, sem2):
    wid = lax.axis_index("s") * 2 + lax.axis_index("c")
    v0 = wid * VOX_PER_W
    e0 = v0 * SEG

    ibufs = (idx_v0, idx_v1, idx_v2)
    wbufs = (w_v0, w_v1, w_v2)
    sems = (sem0, sem1, sem2)

    lanes = lax.iota(jnp.int32, 16)
    zero = jnp.zeros((16,), jnp.float32)
    himask = jnp.full((16,), -65536, jnp.int32)

    maskLo = jnp.where(lanes < 8, 1.0, 0.0).astype(jnp.float32)
    maskHi = jnp.where(lanes < 8, 0.0, 1.0).astype(jnp.float32)
    base17 = lanes * 17

    def fetch(c, s):
        pltpu.async_copy(idx_hbm.at[pl.ds(e0 + c * CHUNK_EL, CHUNK_EL)],
                         ibufs[s], sems[s])
        pltpu.async_copy(w_hbm.at[pl.ds(e0 + c * CHUNK_EL, CHUNK_EL)],
                         wbufs[s], sems[s])

    def triple(s, off):
        si = ibufs[s][pl.ds(off, 16)]
        g = plsc.load_gather(table_v, [si])
        w = wbufs[s][pl.ds(off, 16)]
        x0 = plsc.bitcast(g << 16, jnp.float32)
        x1 = plsc.bitcast(g & himask, jnp.float32)
        return x0 * w, x1 * w

    def mac(s, base, n, a0_init, a1_init):
        @pl.loop(0, n, init_carry=(a0_init, a1_init), unroll=22)
        def r(k, carry):
            a0, a1 = carry
            p0, p1 = triple(s, base + k * 16)
            return a0 + p0, a1 + p1

        return r

    def consume(c, s):
        pltpu.make_async_copy(idx_hbm.at[pl.ds(0, CHUNK_EL)],
                              ibufs[s], sems[s]).wait()
        pltpu.make_async_copy(w_hbm.at[pl.ds(0, CHUNK_EL)],
                              wbufs[s], sems[s]).wait()

        @pl.loop(0, CHUNK_VOX // 2)
        def pairs(p):
            eb = p * (2 * SEG)
            a0, a1 = mac(s, eb, 22, zero, zero)
            xw0, xw1 = triple(s, eb + 352)
            a0 = a0 + xw0 * maskLo
            a1 = a1 + xw1 * maskLo
            b0, b1 = mac(s, eb + 368, 22, xw0 * maskHi, xw1 * maskHi)
            iv_even = base17 + 2 * p
            iv_odd = base17 + (2 * p + 1)
            plsc.store_scatter(ibufs[s], [iv_even],
                               plsc.bitcast(a0, jnp.int32))
            plsc.store_scatter(ibufs[s], [iv_odd],
                               plsc.bitcast(b0, jnp.int32))
            plsc.store_scatter(wbufs[s], [iv_even], a1)
            plsc.store_scatter(wbufs[s], [iv_odd], b1)

        outv0 = zero
        outv1 = zero
        for l in range(16):
            outv0 = outv0 + plsc.bitcast(ibufs[s][pl.ds(17 * l, 16)],
                                         jnp.float32)
            outv1 = outv1 + wbufs[s][pl.ds(17 * l, 16)]
        ro = VOX_PER_W - CHUNK_VOX - c * CHUNK_VOX
        out0_v[pl.ds(ro, CHUNK_VOX)] = (out0_v[pl.ds(ro, CHUNK_VOX)]
                                        + lax.rev(outv0, (0,)) * SCALE)
        out1_v[pl.ds(ro, CHUNK_VOX)] = (out1_v[pl.ds(ro, CHUNK_VOX)]
                                        + lax.rev(outv1, (0,)) * SCALE)

    tcopy = pltpu.async_copy(table_hbm, table_v, tsem)

    pltpu.sync_copy(bias_hbm.at[pl.ds(v0, VOX_PER_W)],
                    w_v0.at[pl.ds(0, VOX_PER_W)])

    @pl.loop(0, NCHUNK)
    def _stage(k):
        rv = lax.rev(w_v0[pl.ds(k * 16, 16)], (0,))
        out0_v[pl.ds(VOX_PER_W - 16 - k * 16, 16)] = rv
        out1_v[pl.ds(VOX_PER_W - 16 - k * 16, 16)] = rv

    fetch(0, 0)
    fetch(1, 1)
    tcopy.wait()

    assert (NCHUNK - 2) % NBUF == 0

    @pl.loop(0, (NCHUNK - 2) // NBUF)
    def _ring(t):
        c = NBUF * t
        for b in range(NBUF):
            fetch(c + b + 2, (b + 2) % NBUF)
            consume(c + b, b)

    c = NCHUNK - 2
    consume(c, c % NBUF)
    consume(c + 1, (c + 1) % NBUF)

    fo = NVOX - v0 - VOX_PER_W
    pltpu.sync_copy(out0_v, out_hbm.at[pl.ds(fo, VOX_PER_W)])
    pltpu.sync_copy(out1_v, out_hbm.at[pl.ds(NVOX + fo, VOX_PER_W)])


_mesh = plsc.VectorSubcoreMesh(core_axis_name="c", subcore_axis_name="s")

_sc_call = pl.kernel(
    _body,
    out_type=jax.ShapeDtypeStruct((B * NVOX,), jnp.float32),
    mesh=_mesh,
    compiler_params=pltpu.CompilerParams(needs_layout_passes=False),
    scratch_types=[
        pltpu.VMEM((SINO,), jnp.int32),
        pltpu.VMEM((CHUNK_EL,), jnp.int32),
        pltpu.VMEM((CHUNK_EL,), jnp.int32),
        pltpu.VMEM((CHUNK_EL,), jnp.int32),
        pltpu.VMEM((CHUNK_EL,), jnp.float32),
        pltpu.VMEM((CHUNK_EL,), jnp.float32),
        pltpu.VMEM((CHUNK_EL,), jnp.float32),
        pltpu.VMEM((VOX_PER_W,), jnp.float32),
        pltpu.VMEM((VOX_PER_W,), jnp.float32),
        pltpu.SemaphoreType.DMA,
        pltpu.SemaphoreType.DMA,
        pltpu.SemaphoreType.DMA,
        pltpu.SemaphoreType.DMA,
    ],
)


@jax.jit
def kernel(input, weight, bias, indices):
    x = input.reshape(B, SINO)
    lo = lax.bitcast_convert_type(x[0].astype(jnp.bfloat16), jnp.uint16)
    hi = lax.bitcast_convert_type(x[1].astype(jnp.bfloat16), jnp.uint16)
    table = (lo.astype(jnp.uint32) | (hi.astype(jnp.uint32) << 16)
             ).view(jnp.int32)
    out = _sc_call(table, indices, weight.reshape(-1), bias)
    return out.reshape(B, 1, NVX, NVY)
